# baseline (speedup 1.0000x reference)
import functools

import jax
import jax.numpy as jnp
from jax import lax
from jax.experimental import pallas as pl
from jax.experimental.pallas import tpu as pltpu
from jax.experimental.pallas import tpu_sc as plsc

_N = 10000
_E = 320000
_H = 128
_NUM_DEG = 200
_NW = 32
_EPT = _E // _NW
_NCH = _EPT // 16
_NP = 10240
_RS = _NP // 16

_BLK = 5000
_NB = _N // _BLK
_EB = _E // _BLK


def _sc_body(src1, dst1, src2, dst2, coord1, coord2,
             d_out, dec_out, deg_out,
             src_v, dst_v, xyz_v, d_v, dec_v,
             hist_v, tmp_v, acc_v, hist_sh):
  c = lax.axis_index("c")
  s = lax.axis_index("s")
  wid = s * 2 + c
  base = wid * _EPT
  ones16 = jnp.ones((16,), jnp.int32)
  zeros16 = jnp.zeros((16,), jnp.int32)

  for g, (src, dst, coord) in enumerate(((src1, dst1, coord1),
                                         (src2, dst2, coord2))):
    pltpu.sync_copy(src.at[pl.ds(base, _EPT)], src_v)
    pltpu.sync_copy(dst.at[pl.ds(base, _EPT)], dst_v)
    pltpu.sync_copy(coord, xyz_v)

    def zbody(j, carry):
      hist_v[pl.ds(j * 16, 16)] = zeros16
      return carry
    lax.fori_loop(0, _NP // 16, zbody, 0)

    def ebody(j, carry):
      sl = pl.ds(j * 16, 16)
      si = src_v[sl]
      di = dst_v[sl]
      xs = plsc.load_gather(xyz_v, [si])
      ys = plsc.load_gather(xyz_v, [si + _N])
      zs = plsc.load_gather(xyz_v, [si + 2 * _N])
      xd = plsc.load_gather(xyz_v, [di])
      yd = plsc.load_gather(xyz_v, [di + _N])
      zd = plsc.load_gather(xyz_v, [di + 2 * _N])
      dx = xs - xd
      dy = ys - yd
      dz = zs - zd
      d2 = dx * dx + dy * dy + dz * dz
      mi = jnp.int32(0x5F3759DF) - lax.shift_right_arithmetic(
          plsc.bitcast(d2, jnp.int32), 1)
      r = plsc.bitcast(mi, jnp.float32)
      for _ in range(3):
        r = r * (1.5 - 0.5 * d2 * r * r)
      d = d2 * r
      d_v[sl] = d
      dm = d - 2.0
      e = jnp.exp(dm * dm * jnp.float32(-1.0 / 1.100001))
      dec_v[sl] = jnp.where(d > 2.0, e, jnp.float32(1.0))
      plsc.addupdate_scatter(hist_v, [di], ones16)
      return carry
    lax.fori_loop(0, _NCH, ebody, 0)

    pltpu.sync_copy(d_v, d_out.at[pl.ds(g * _E + base, _EPT)])
    pltpu.sync_copy(dec_v, dec_out.at[pl.ds(g * _E + base, _EPT)])
    pltpu.sync_copy(hist_v, hist_sh.at[pl.ds((g * 16 + s) * _NP, _NP)])

  plsc.subcore_barrier()

  for g in range(2):
    for t in range(16):
      pltpu.sync_copy(hist_sh.at[pl.ds((g * 16 + t) * _NP + s * _RS, _RS)],
                      tmp_v.at[pl.ds(t * _RS, _RS)])

    def rbody(j, carry):
      acc = tmp_v[pl.ds(j * 16, 16)]
      for t in range(1, 16):
        acc = acc + tmp_v[pl.ds(t * _RS + j * 16, 16)]
      acc_v[pl.ds(j * 16, 16)] = acc
      return carry
    lax.fori_loop(0, _RS // 16, rbody, 0)

    pltpu.sync_copy(acc_v,
                    deg_out.at[pl.ds((c * 2 + g) * _NP + s * _RS, _RS)])


def _sc_call(src1, dst1, src2, dst2, coord1, coord2):
  mesh = plsc.VectorSubcoreMesh(core_axis_name="c", subcore_axis_name="s",
                                num_cores=2, num_subcores=16)
  fn = pl.kernel(
      _sc_body,
      out_type=[
          jax.ShapeDtypeStruct((2 * _E,), jnp.float32),
          jax.ShapeDtypeStruct((2 * _E,), jnp.float32),
          jax.ShapeDtypeStruct((4 * _NP,), jnp.int32),
      ],
      mesh=mesh,
      scratch_types=[
          pltpu.VMEM((_EPT,), jnp.int32),
          pltpu.VMEM((_EPT,), jnp.int32),
          pltpu.VMEM((3 * _N,), jnp.float32),
          pltpu.VMEM((_EPT,), jnp.float32),
          pltpu.VMEM((_EPT,), jnp.float32),
          pltpu.VMEM((_NP,), jnp.int32),
          pltpu.VMEM((16 * _RS,), jnp.int32),
          pltpu.VMEM((_RS,), jnp.int32),
          pltpu.VMEM_SHARED((2 * 16 * _NP,), jnp.int32),
      ],
      compiler_params=pltpu.CompilerParams(needs_layout_passes=False),
      name="dmpnn_sc_gather",
  )
  return fn(src1, dst1, src2, dst2, coord1, coord2)


def _layernorm(x, g, b):
  o = jnp.full((_H, _H), 1.0 / _H, jnp.float32)
  m = jnp.dot(x, o, preferred_element_type=jnp.float32)
  xc = x - m
  v = jnp.dot(xc * xc, o, preferred_element_type=jnp.float32)
  return xc * lax.rsqrt(v + 1e-5) * g + b


def _tc_body(af1_ref, af2_ref, ef1_ref, ef2_ref, d_ref, dec_ref,
             p0_ref, p1_ref, wa_ref, ba_ref, wep_ref, wrp_ref, bc_ref,
             lng_ref, lnb_ref, demb_ref, out_ref):
  i = pl.program_id(0)

  def node_out(af_ref):
    x = af_ref[...]
    h = jax.nn.relu(
        jnp.dot(x, wa_ref[...], preferred_element_type=jnp.float32)
        + ba_ref[...])
    deg = jnp.clip(p0_ref[...] + p1_ref[...], 0, _NUM_DEG - 1)
    oh = (lax.broadcasted_iota(jnp.int32, (_BLK, _NUM_DEG), 1)
          == deg).astype(jnp.float32)
    h = h + jnp.dot(oh, demb_ref[...], preferred_element_type=jnp.float32)
    return _layernorm(h, lng_ref[...], lnb_ref[...])

  def edge_out(ef_ref):
    ef = ef_ref[...]
    d = d_ref[...]
    mu = lax.broadcasted_iota(jnp.int32, (1, 16), 1).astype(jnp.float32) * (
        5.0 / 15.0)
    rbf = jnp.exp(-jnp.square((d - mu) * (16.0 / 5.0)))
    he = jax.nn.relu(
        jnp.dot(ef, wep_ref[...], preferred_element_type=jnp.float32)
        + jnp.dot(rbf, wrp_ref[...], preferred_element_type=jnp.float32)
        + bc_ref[...])
    return _layernorm(he, lng_ref[...], lnb_ref[...]) * dec_ref[...]

  @pl.when(i < _NB)
  def _():
    out_ref[...] = node_out(af1_ref)

  @pl.when(jnp.logical_and(i >= _NB, i < _NB + _EB))
  def _():
    out_ref[...] = edge_out(ef1_ref)

  @pl.when(jnp.logical_and(i >= _NB + _EB, i < 2 * _NB + _EB))
  def _():
    out_ref[...] = node_out(af2_ref)

  @pl.when(i >= 2 * _NB + _EB)
  def _():
    out_ref[...] = edge_out(ef2_ref)


def _tc_call(af1, af2, ef1, ef2, dcol, deccol, p0, p1,
             wa, ba, wep, wrp, bc, lng, lnb, demb):
  nsteps = 2 * (_NB + _EB)
  half = _NB + _EB

  def full(a):
    return pl.BlockSpec(a.shape, lambda i: (0,) * a.ndim)

  def edge_row(i):
    return (jnp.where(i < half, jnp.clip(i - _NB, 0, _EB - 1),
                      jnp.clip(i - 2 * _NB, _EB, 2 * _EB - 1)), 0)

  def node_row(i):
    return (jnp.where(i < half, jnp.clip(i, 0, _NB - 1),
                      jnp.clip(i - _EB, _NB, 2 * _NB - 1)), 0)

  grid_spec = pl.GridSpec(
      grid=(nsteps,),
      in_specs=[
          pl.BlockSpec((_BLK, 70), lambda i: (jnp.minimum(i, _NB - 1), 0)),
          pl.BlockSpec((_BLK, 70), lambda i: (jnp.clip(i - half, 0, _NB - 1), 0)),
          pl.BlockSpec((_BLK, 14), lambda i: (jnp.clip(i - _NB, 0, _EB - 1), 0)),
          pl.BlockSpec((_BLK, 14),
                       lambda i: (jnp.clip(i - half - _NB, 0, _EB - 1), 0)),
          pl.BlockSpec((_BLK, 1), edge_row),
          pl.BlockSpec((_BLK, 1), edge_row),
          pl.BlockSpec((_BLK, 1), node_row),
          pl.BlockSpec((_BLK, 1), node_row),
          full(wa), full(ba), full(wep), full(wrp), full(bc),
          full(lng), full(lnb), full(demb),
      ],
      out_specs=pl.BlockSpec((_BLK, _H), lambda i: (i, 0)),
  )
  return pl.pallas_call(
      _tc_body,
      grid_spec=grid_spec,
      out_shape=jax.ShapeDtypeStruct((2 * (_N + _E), _H), jnp.float32),
      name="dmpnn_tc_fused",
  )(af1, af2, ef1, ef2, dcol, deccol, p0, p1,
    wa, ba, wep, wrp, bc, lng, lnb, demb)


def kernel(atom_feature1, atom_coordinate1, edge_feature1, edge_index1,
           atom_feature2, atom_coordinate2, edge_feature2, edge_index2,
           W_atom, b_atom, W_edge, W_rbf, b_rbf, ln_g, ln_b, deg_emb):
  f32 = jnp.float32
  coord1 = jnp.transpose(atom_coordinate1).reshape(3 * _N)
  coord2 = jnp.transpose(atom_coordinate2).reshape(3 * _N)

  d, dec, degf = _sc_call(edge_index1[0], edge_index1[1],
                          edge_index2[0], edge_index2[1], coord1, coord2)
  dcol = d.reshape(2 * _E, 1)
  deccol = dec.reshape(2 * _E, 1)
  degp = degf.reshape(2, 2, _NP)
  p0 = degp[0, :, :_N].reshape(2 * _N, 1)
  p1 = degp[1, :, :_N].reshape(2 * _N, 1)

  hh = _H // 2
  wep = jnp.concatenate([W_edge, jnp.zeros((14, hh), f32)], axis=1)
  wrp = jnp.concatenate([jnp.zeros((16, hh), f32), W_rbf], axis=1)
  bc = jnp.concatenate([jnp.zeros((hh,), f32), b_rbf]).reshape(1, _H)

  return _tc_call(
      atom_feature1, atom_feature2, edge_feature1, edge_feature2,
      dcol, deccol, p0, p1,
      W_atom, b_atom.reshape(1, _H), wep, wrp, bc,
      ln_g.reshape(1, _H), ln_b.reshape(1, _H), deg_emb)


# speedup vs baseline: 1.0167x; 1.0167x over previous
import functools

import jax
import jax.numpy as jnp
from jax import lax
from jax.experimental import pallas as pl
from jax.experimental.pallas import tpu as pltpu
from jax.experimental.pallas import tpu_sc as plsc

_N = 10000
_E = 320000
_H = 128
_NUM_DEG = 200
_NW = 32
_EPT = _E // _NW
_NCH = _EPT // 16
_NP = 10240
_RS = _NP // 16

_BLK = 5000
_NB = _N // _BLK
_EB = _E // _BLK


def _sc_body(src1, dst1, src2, dst2, coord1, coord2,
             d_out, dec_out, deg_out,
             src_v, dst_v, xyz_v, d_v, dec_v,
             hist_v, tmp_v, acc_v, hist_sh):
  c = lax.axis_index("c")
  s = lax.axis_index("s")
  wid = s * 2 + c
  base = wid * _EPT
  ones16 = jnp.ones((16,), jnp.int32)
  zeros16 = jnp.zeros((16,), jnp.int32)

  for g, (src, dst, coord) in enumerate(((src1, dst1, coord1),
                                         (src2, dst2, coord2))):
    pltpu.sync_copy(src.at[pl.ds(base, _EPT)], src_v)
    pltpu.sync_copy(dst.at[pl.ds(base, _EPT)], dst_v)
    pltpu.sync_copy(coord, xyz_v)

    def zbody(j, carry):
      hist_v[pl.ds(j * 16, 16)] = zeros16
      return carry
    lax.fori_loop(0, _NP // 16, zbody, 0)

    def ebody(j, carry):
      sl = pl.ds(j * 16, 16)
      si = src_v[sl]
      di = dst_v[sl]
      xs = plsc.load_gather(xyz_v, [si])
      ys = plsc.load_gather(xyz_v, [si + _N])
      zs = plsc.load_gather(xyz_v, [si + 2 * _N])
      xd = plsc.load_gather(xyz_v, [di])
      yd = plsc.load_gather(xyz_v, [di + _N])
      zd = plsc.load_gather(xyz_v, [di + 2 * _N])
      dx = xs - xd
      dy = ys - yd
      dz = zs - zd
      d2 = dx * dx + dy * dy + dz * dz
      mi = jnp.int32(0x5F3759DF) - lax.shift_right_arithmetic(
          plsc.bitcast(d2, jnp.int32), 1)
      r = plsc.bitcast(mi, jnp.float32)
      for _ in range(3):
        r = r * (1.5 - 0.5 * d2 * r * r)
      d = d2 * r
      d_v[sl] = d
      dm = d - 2.0
      e = jnp.exp(dm * dm * jnp.float32(-1.0 / 1.100001))
      dec_v[sl] = jnp.where(d > 2.0, e, jnp.float32(1.0))
      plsc.addupdate_scatter(hist_v, [di], ones16)
      return carry
    lax.fori_loop(0, _NCH, ebody, 0)

    pltpu.sync_copy(d_v, d_out.at[pl.ds(g * _E + base, _EPT)])
    pltpu.sync_copy(dec_v, dec_out.at[pl.ds(g * _E + base, _EPT)])
    pltpu.sync_copy(hist_v, hist_sh.at[pl.ds((g * 16 + s) * _NP, _NP)])

  plsc.subcore_barrier()

  for g in range(2):
    for t in range(16):
      pltpu.sync_copy(hist_sh.at[pl.ds((g * 16 + t) * _NP + s * _RS, _RS)],
                      tmp_v.at[pl.ds(t * _RS, _RS)])

    def rbody(j, carry):
      acc = tmp_v[pl.ds(j * 16, 16)]
      for t in range(1, 16):
        acc = acc + tmp_v[pl.ds(t * _RS + j * 16, 16)]
      acc_v[pl.ds(j * 16, 16)] = acc
      return carry
    lax.fori_loop(0, _RS // 16, rbody, 0)

    pltpu.sync_copy(acc_v,
                    deg_out.at[pl.ds((c * 2 + g) * _NP + s * _RS, _RS)])


def _sc_call(src1, dst1, src2, dst2, coord1, coord2):
  mesh = plsc.VectorSubcoreMesh(core_axis_name="c", subcore_axis_name="s",
                                num_cores=2, num_subcores=16)
  fn = pl.kernel(
      _sc_body,
      out_type=[
          jax.ShapeDtypeStruct((2 * _E,), jnp.float32),
          jax.ShapeDtypeStruct((2 * _E,), jnp.float32),
          jax.ShapeDtypeStruct((4 * _NP,), jnp.int32),
      ],
      mesh=mesh,
      scratch_types=[
          pltpu.VMEM((_EPT,), jnp.int32),
          pltpu.VMEM((_EPT,), jnp.int32),
          pltpu.VMEM((3 * _N,), jnp.float32),
          pltpu.VMEM((_EPT,), jnp.float32),
          pltpu.VMEM((_EPT,), jnp.float32),
          pltpu.VMEM((_NP,), jnp.int32),
          pltpu.VMEM((16 * _RS,), jnp.int32),
          pltpu.VMEM((_RS,), jnp.int32),
          pltpu.VMEM_SHARED((2 * 16 * _NP,), jnp.int32),
      ],
      compiler_params=pltpu.CompilerParams(needs_layout_passes=False),
      name="dmpnn_sc_gather",
  )
  return fn(src1, dst1, src2, dst2, coord1, coord2)


def _layernorm(x, g, b):
  m = jnp.mean(x, axis=-1, keepdims=True)
  xc = x - m
  v = jnp.mean(xc * xc, axis=-1, keepdims=True)
  return xc * lax.rsqrt(v + 1e-5) * g + b


def _tc_body(af1_ref, af2_ref, ef1_ref, ef2_ref, d_ref, dec_ref,
             p0_ref, p1_ref, wa_ref, ba_ref, wep_ref, wrp_ref, bc_ref,
             lng_ref, lnb_ref, demb_ref, out_ref):
  i = pl.program_id(0)

  def node_out(af_ref):
    x = af_ref[...]
    h = jax.nn.relu(
        jnp.dot(x, wa_ref[...], preferred_element_type=jnp.float32)
        + ba_ref[...])
    deg = jnp.clip(p0_ref[...] + p1_ref[...], 0, _NUM_DEG - 1)
    oh = (lax.broadcasted_iota(jnp.int32, (_BLK, _NUM_DEG), 1)
          == deg).astype(jnp.float32)
    h = h + jnp.dot(oh, demb_ref[...], preferred_element_type=jnp.float32)
    return _layernorm(h, lng_ref[...], lnb_ref[...])

  def edge_out(ef_ref):
    ef = ef_ref[...]
    d = d_ref[...]
    mu = lax.broadcasted_iota(jnp.int32, (1, 16), 1).astype(jnp.float32) * (
        5.0 / 15.0)
    rbf = jnp.exp(-jnp.square((d - mu) * (16.0 / 5.0)))
    he = jax.nn.relu(
        jnp.dot(ef, wep_ref[...], preferred_element_type=jnp.float32)
        + jnp.dot(rbf, wrp_ref[...], preferred_element_type=jnp.float32)
        + bc_ref[...])
    return _layernorm(he, lng_ref[...], lnb_ref[...]) * dec_ref[...]

  @pl.when(i < _NB)
  def _():
    out_ref[...] = node_out(af1_ref)

  @pl.when(jnp.logical_and(i >= _NB, i < _NB + _EB))
  def _():
    out_ref[...] = edge_out(ef1_ref)

  @pl.when(jnp.logical_and(i >= _NB + _EB, i < 2 * _NB + _EB))
  def _():
    out_ref[...] = node_out(af2_ref)

  @pl.when(i >= 2 * _NB + _EB)
  def _():
    out_ref[...] = edge_out(ef2_ref)


def _tc_call(af1, af2, ef1, ef2, dcol, deccol, p0, p1,
             wa, ba, wep, wrp, bc, lng, lnb, demb):
  nsteps = 2 * (_NB + _EB)
  half = _NB + _EB

  def full(a):
    return pl.BlockSpec(a.shape, lambda i: (0,) * a.ndim)

  def edge_row(i):
    return (jnp.where(i < half, jnp.clip(i - _NB, 0, _EB - 1),
                      jnp.clip(i - 2 * _NB, _EB, 2 * _EB - 1)), 0)

  def node_row(i):
    return (jnp.where(i < half, jnp.clip(i, 0, _NB - 1),
                      jnp.clip(i - _EB, _NB, 2 * _NB - 1)), 0)

  grid_spec = pl.GridSpec(
      grid=(nsteps,),
      in_specs=[
          pl.BlockSpec((_BLK, 70), lambda i: (jnp.minimum(i, _NB - 1), 0)),
          pl.BlockSpec((_BLK, 70), lambda i: (jnp.clip(i - half, 0, _NB - 1), 0)),
          pl.BlockSpec((_BLK, 14), lambda i: (jnp.clip(i - _NB, 0, _EB - 1), 0)),
          pl.BlockSpec((_BLK, 14),
                       lambda i: (jnp.clip(i - half - _NB, 0, _EB - 1), 0)),
          pl.BlockSpec((_BLK, 1), edge_row),
          pl.BlockSpec((_BLK, 1), edge_row),
          pl.BlockSpec((_BLK, 1), node_row),
          pl.BlockSpec((_BLK, 1), node_row),
          full(wa), full(ba), full(wep), full(wrp), full(bc),
          full(lng), full(lnb), full(demb),
      ],
      out_specs=pl.BlockSpec((_BLK, _H), lambda i: (i, 0)),
  )
  return pl.pallas_call(
      _tc_body,
      grid_spec=grid_spec,
      out_shape=jax.ShapeDtypeStruct((2 * (_N + _E), _H), jnp.float32),
      name="dmpnn_tc_fused",
  )(af1, af2, ef1, ef2, dcol, deccol, p0, p1,
    wa, ba, wep, wrp, bc, lng, lnb, demb)


def kernel(atom_feature1, atom_coordinate1, edge_feature1, edge_index1,
           atom_feature2, atom_coordinate2, edge_feature2, edge_index2,
           W_atom, b_atom, W_edge, W_rbf, b_rbf, ln_g, ln_b, deg_emb):
  f32 = jnp.float32
  coord1 = jnp.transpose(atom_coordinate1).reshape(3 * _N)
  coord2 = jnp.transpose(atom_coordinate2).reshape(3 * _N)

  d, dec, degf = _sc_call(edge_index1[0], edge_index1[1],
                          edge_index2[0], edge_index2[1], coord1, coord2)
  dcol = d.reshape(2 * _E, 1)
  deccol = dec.reshape(2 * _E, 1)
  degp = degf.reshape(2, 2, _NP)
  p0 = degp[0, :, :_N].reshape(2 * _N, 1)
  p1 = degp[1, :, :_N].reshape(2 * _N, 1)

  hh = _H // 2
  wep = jnp.concatenate([W_edge, jnp.zeros((14, hh), f32)], axis=1)
  wrp = jnp.concatenate([jnp.zeros((16, hh), f32), W_rbf], axis=1)
  bc = jnp.concatenate([jnp.zeros((hh,), f32), b_rbf]).reshape(1, _H)

  return _tc_call(
      atom_feature1, atom_feature2, edge_feature1, edge_feature2,
      dcol, deccol, p0, p1,
      W_atom, b_atom.reshape(1, _H), wep, wrp, bc,
      ln_g.reshape(1, _H), ln_b.reshape(1, _H), deg_emb)


# speedup vs baseline: 1.2224x; 1.2023x over previous
import functools

import jax
import jax.numpy as jnp
from jax import lax
from jax.experimental import pallas as pl
from jax.experimental.pallas import tpu as pltpu
from jax.experimental.pallas import tpu_sc as plsc

_N = 10000
_E = 320000
_H = 128
_NUM_DEG = 200
_NW = 32
_EPT = _E // _NW
_NCH = _EPT // 16
_NP = 10240
_RS = _NP // 16
_NROW = 80
_EPAD = _NROW * 128 - _EPT

_BLK = 5000
_NB = _N // _BLK
_EB = _E // _BLK


def _sc_body(src1, dst1, src2, dst2, coord1, coord2,
             dw_out, deg_out,
             src_v, dst_v, xyz_v, d_v,
             hist_v, tmp_v, acc_v, hist_sh):
  c = lax.axis_index("c")
  s = lax.axis_index("s")
  wid = s * 2 + c
  base = wid * _EPT
  ones16 = jnp.ones((16,), jnp.int32)
  zeros16 = jnp.zeros((16,), jnp.int32)

  for g, (src, dst, coord) in enumerate(((src1, dst1, coord1),
                                         (src2, dst2, coord2))):
    pltpu.sync_copy(src.at[pl.ds(base, _EPT)], src_v)
    pltpu.sync_copy(dst.at[pl.ds(base, _EPT)], dst_v)
    pltpu.sync_copy(coord, xyz_v)

    def zbody(j, carry):
      hist_v[pl.ds(j * 16, 16)] = zeros16
      return carry
    lax.fori_loop(0, _NP // 16, zbody, 0)

    def ebody(j, carry):
      sl = pl.ds(j * 16, 16)
      si = src_v[sl]
      di = dst_v[sl]
      xs = plsc.load_gather(xyz_v, [si])
      ys = plsc.load_gather(xyz_v, [si + _N])
      zs = plsc.load_gather(xyz_v, [si + 2 * _N])
      xd = plsc.load_gather(xyz_v, [di])
      yd = plsc.load_gather(xyz_v, [di + _N])
      zd = plsc.load_gather(xyz_v, [di + 2 * _N])
      dx = xs - xd
      dy = ys - yd
      dz = zs - zd
      d2 = dx * dx + dy * dy + dz * dz
      mi = jnp.int32(0x5F3759DF) - lax.shift_right_arithmetic(
          plsc.bitcast(d2, jnp.int32), 1)
      r = plsc.bitcast(mi, jnp.float32)
      for _ in range(3):
        r = r * (1.5 - 0.5 * d2 * r * r)
      d_v[sl] = d2 * r
      plsc.addupdate_scatter(hist_v, [di], ones16)
      return carry
    lax.fori_loop(0, _NCH, ebody, 0)

    pltpu.sync_copy(d_v.at[pl.ds(0, _EPT)],
                    dw_out.at[pl.ds(g * _E + base, _EPT)])
    pltpu.sync_copy(hist_v, hist_sh.at[pl.ds((g * 16 + s) * _NP, _NP)])

  plsc.subcore_barrier()

  for g in range(2):
    for t in range(16):
      pltpu.sync_copy(hist_sh.at[pl.ds((g * 16 + t) * _NP + s * _RS, _RS)],
                      tmp_v.at[pl.ds(t * _RS, _RS)])

    def rbody(j, carry):
      acc = tmp_v[pl.ds(j * 16, 16)]
      for t in range(1, 16):
        acc = acc + tmp_v[pl.ds(t * _RS + j * 16, 16)]
      acc_v[pl.ds(j * 16, 16)] = acc
      return carry
    lax.fori_loop(0, _RS // 16, rbody, 0)

    pltpu.sync_copy(acc_v,
                    deg_out.at[pl.ds((c * 2 + g) * _NP + s * _RS, _RS)])


def _sc_call(src1, dst1, src2, dst2, coord1, coord2):
  mesh = plsc.VectorSubcoreMesh(core_axis_name="c", subcore_axis_name="s",
                                num_cores=2, num_subcores=16)
  fn = pl.kernel(
      _sc_body,
      out_type=[
          jax.ShapeDtypeStruct((2 * _E,), jnp.float32),
          jax.ShapeDtypeStruct((4 * _NP,), jnp.int32),
      ],
      mesh=mesh,
      scratch_types=[
          pltpu.VMEM((_EPT,), jnp.int32),
          pltpu.VMEM((_EPT,), jnp.int32),
          pltpu.VMEM((3 * _N,), jnp.float32),
          pltpu.VMEM((_EPT,), jnp.float32),
          pltpu.VMEM((_NP,), jnp.int32),
          pltpu.VMEM((16 * _RS,), jnp.int32),
          pltpu.VMEM((_RS,), jnp.int32),
          pltpu.VMEM_SHARED((2 * 16 * _NP,), jnp.int32),
      ],
      compiler_params=pltpu.CompilerParams(needs_layout_passes=False),
      name="dmpnn_sc_gather",
  )
  return fn(src1, dst1, src2, dst2, coord1, coord2)


def _layernorm(x, g, b):
  m = jnp.mean(x, axis=-1, keepdims=True)
  xc = x - m
  v = jnp.mean(xc * xc, axis=-1, keepdims=True)
  return xc * lax.rsqrt(v + 1e-5) * g + b


def _tc_body(af1_ref, af2_ref, ef1_ref, ef2_ref, d_ref,
             p0_ref, p1_ref, wa_ref, ba_ref, wep_ref, wrp_ref, bc_ref,
             lng_ref, lnb_ref, demb_ref, out_ref):
  i = pl.program_id(0)

  def node_out(af_ref):
    x = af_ref[...]
    h = jax.nn.relu(
        jnp.dot(x, wa_ref[...], preferred_element_type=jnp.float32)
        + ba_ref[...])
    deg = jnp.clip(p0_ref[...] + p1_ref[...], 0, _NUM_DEG - 1)
    oh = (lax.broadcasted_iota(jnp.int32, (_BLK, _NUM_DEG), 1)
          == deg).astype(jnp.float32)
    h = h + jnp.dot(oh, demb_ref[...], preferred_element_type=jnp.float32)
    return _layernorm(h, lng_ref[...], lnb_ref[...])

  def edge_out(ef_ref, eb):
    ef = ef_ref[...]
    d = d_ref[...]
    dec = jnp.where(
        d > 2.0, jnp.exp(-jnp.square(d - 2.0) / (1.1 + 1e-06)),
        jnp.float32(1.0))
    mu = lax.broadcasted_iota(jnp.int32, (1, 16), 1).astype(jnp.float32) * (
        5.0 / 15.0)
    rbf = jnp.exp(-jnp.square((d - mu) * (16.0 / 5.0)))
    he = jax.nn.relu(
        jnp.dot(ef, wep_ref[...], preferred_element_type=jnp.float32)
        + jnp.dot(rbf, wrp_ref[...], preferred_element_type=jnp.float32)
        + bc_ref[...])
    return _layernorm(he, lng_ref[...], lnb_ref[...]) * dec

  @pl.when(i < _NB)
  def _():
    out_ref[...] = node_out(af1_ref)

  @pl.when(jnp.logical_and(i >= _NB, i < _NB + _EB))
  def _():
    out_ref[...] = edge_out(ef1_ref, i - _NB)

  @pl.when(jnp.logical_and(i >= _NB + _EB, i < 2 * _NB + _EB))
  def _():
    out_ref[...] = node_out(af2_ref)

  @pl.when(i >= 2 * _NB + _EB)
  def _():
    out_ref[...] = edge_out(ef2_ref, i - 2 * _NB)


def _tc_call(af1, af2, ef1, ef2, dwide, p0, p1,
             wa, ba, wep, wrp, bc, lng, lnb, demb):
  nsteps = 2 * (_NB + _EB)
  half = _NB + _EB

  def full(a):
    return pl.BlockSpec(a.shape, lambda i: (0,) * a.ndim)

  def edge_row(i):
    return (jnp.where(i < half, jnp.clip(i - _NB, 0, _EB - 1),
                      jnp.clip(i - 2 * _NB, _EB, 2 * _EB - 1)), 0)

  def node_row(i):
    return (jnp.where(i < half, jnp.clip(i, 0, _NB - 1),
                      jnp.clip(i - _EB, _NB, 2 * _NB - 1)), 0)

  in_specs = [
          pl.BlockSpec((_BLK, 70), lambda i: (jnp.minimum(i, _NB - 1), 0)),
          pl.BlockSpec((_BLK, 70), lambda i: (jnp.clip(i - half, 0, _NB - 1), 0)),
          pl.BlockSpec((_BLK, 14), lambda i: (jnp.clip(i - _NB, 0, _EB - 1), 0)),
          pl.BlockSpec((_BLK, 14),
                       lambda i: (jnp.clip(i - half - _NB, 0, _EB - 1), 0)),
          pl.BlockSpec((_BLK, 1), edge_row),
          pl.BlockSpec((_BLK, 1), node_row),
          pl.BlockSpec((_BLK, 1), node_row),
          full(wa), full(ba), full(wep), full(wrp), full(bc),
          full(lng), full(lnb), full(demb),
  ]
  return pl.pallas_call(
      _tc_body,
      grid=(nsteps,),
      in_specs=in_specs,
      out_specs=pl.BlockSpec((_BLK, _H), lambda i: (i, 0)),
      out_shape=jax.ShapeDtypeStruct((2 * (_N + _E), _H), jnp.float32),
      name="dmpnn_tc_fused",
  )(af1, af2, ef1, ef2, dwide, p0, p1,
    wa, ba, wep, wrp, bc, lng, lnb, demb)


def kernel(atom_feature1, atom_coordinate1, edge_feature1, edge_index1,
           atom_feature2, atom_coordinate2, edge_feature2, edge_index2,
           W_atom, b_atom, W_edge, W_rbf, b_rbf, ln_g, ln_b, deg_emb):
  f32 = jnp.float32
  coord1 = jnp.transpose(atom_coordinate1).reshape(3 * _N)
  coord2 = jnp.transpose(atom_coordinate2).reshape(3 * _N)

  dw, degf = _sc_call(edge_index1[0], edge_index1[1],
                      edge_index2[0], edge_index2[1], coord1, coord2)
  dwide = dw.reshape(2 * _E, 1)
  degp = degf.reshape(2, 2, _NP)
  p0 = degp[0, :, :_N].reshape(2 * _N, 1)
  p1 = degp[1, :, :_N].reshape(2 * _N, 1)

  hh = _H // 2
  wep = jnp.concatenate([W_edge, jnp.zeros((14, hh), f32)], axis=1)
  wrp = jnp.concatenate([jnp.zeros((16, hh), f32), W_rbf], axis=1)
  bc = jnp.concatenate([jnp.zeros((hh,), f32), b_rbf]).reshape(1, _H)

  return _tc_call(
      atom_feature1, atom_feature2, edge_feature1, edge_feature2,
      dwide, p0, p1,
      W_atom, b_atom.reshape(1, _H), wep, wrp, bc,
      ln_g.reshape(1, _H), ln_b.reshape(1, _H), deg_emb)
